# baseline (speedup 1.0000x reference)
import functools

import jax
import jax.numpy as jnp
from jax import lax
from jax.experimental import pallas as pl
from jax.experimental.pallas import tpu as pltpu
from jax.experimental.pallas import tpu_sc as plsc

N = 16384
V = 1000000
D = 64
T_PAD = 64
CH = 128


def _make_kernel():
    info = plsc.get_sparse_core_info()
    NC, NS, L = info.num_cores, info.num_subcores, info.num_lanes
    NW = NC * NS
    BPW = N // NW
    NCH = BPW // CH

    mesh = plsc.VectorSubcoreMesh(core_axis_name="c", subcore_axis_name="s")

    @functools.partial(
        pl.kernel,
        mesh=mesh,
        out_type=jax.ShapeDtypeStruct((N, D), jnp.float32),
        compiler_params=pltpu.CompilerParams(needs_layout_passes=False),
        scratch_types=[
            pltpu.VMEM((BPW,), jnp.int32),
            pltpu.VMEM((BPW,), jnp.int32),
            pltpu.VMEM((CH, 2 * D), jnp.float32),
            pltpu.VMEM((CH, D), jnp.float32),
            pltpu.VMEM((T_PAD * T_PAD,), jnp.float32),
            pltpu.VMEM((T_PAD,), jnp.float32),
            pltpu.VMEM((BPW,), jnp.int32),
            pltpu.VMEM((BPW,), jnp.float32),
            pltpu.VMEM((BPW,), jnp.float32),
            pltpu.SemaphoreType.DMA,
        ],
    )
    def emb_kernel(x_hbm, maskf_hbm, tid_hbm, tag_hbm, emb_hbm, out_hbm,
                   idx_v, sidx_v, pair_v, stage_v, tag_v, sums_v, tid_v,
                   maskf_v, scale_v, gsem):
        wid = lax.axis_index("s") * NC + lax.axis_index("c")
        base = wid * BPW

        pltpu.sync_copy(x_hbm.at[pl.ds(base, BPW)], idx_v)
        for g in range(BPW // L):
            sl = pl.ds(g * L, L)
            sidx_v[sl] = lax.shift_right_logical(idx_v[sl], 1)

        def fire(c):
            pltpu.async_copy(emb_hbm.at[sidx_v.at[pl.ds(c * CH, CH)]],
                             pair_v, gsem)

        fire(0)

        pltpu.sync_copy(tag_hbm, tag_v)
        pltpu.sync_copy(tid_hbm.at[pl.ds(base, BPW)], tid_v)
        pltpu.sync_copy(maskf_hbm.at[pl.ds(base, BPW)], maskf_v)

        lanes = lax.iota(jnp.int32, L)
        for g in range(T_PAD // L):
            t_vec = lanes + (g * L)
            row_base = t_vec * T_PAD
            acc = jnp.zeros((L,), jnp.float32)
            for c in range(T_PAD):
                acc = acc + plsc.load_gather(tag_v, [row_base + c])
            plsc.store_scatter(sums_v, [t_vec], acc)

        for g in range(BPW // L):
            sl = pl.ds(g * L, L)
            scale_v[sl] = plsc.load_gather(sums_v, [tid_v[sl]]) * maskf_v[sl]

        for c in range(NCH):
            pltpu.make_async_copy(emb_hbm.at[sidx_v.at[pl.ds(0, CH)]],
                                  pair_v, gsem).wait()

            def extract(g16, _, c=c):
                n0 = c * CH + g16 * L
                hv = jnp.bitwise_and(idx_v[pl.ds(n0, L)], 1)
                sv = scale_v[pl.ds(n0, L)]
                for i in range(L):
                    k = g16 * L + i
                    sb = jnp.full((L,), sv[i], jnp.float32)
                    off = hv[i] * D
                    for j in range(D // L):
                        stage_v[k, pl.ds(j * L, L)] = (
                            pair_v[k, pl.ds(off + j * L, L)] * sb)
                return _

            lax.fori_loop(0, CH // L, extract, None)
            pltpu.sync_copy(stage_v, out_hbm.at[pl.ds(base + c * CH, CH)])
            if c + 1 < NCH:
                fire(c + 1)

    return emb_kernel


_emb_kernel = _make_kernel()


@jax.jit
def kernel(x, mask, tag_id, emb_table, tag_table):
    x = x.astype(jnp.int32)
    maskf = mask.astype(jnp.float32)
    tag_id = tag_id.astype(jnp.int32)
    t, td = tag_table.shape
    tag_pad = jnp.zeros((T_PAD, T_PAD), jnp.float32).at[:t, :td].set(tag_table)
    emb_pairs = emb_table.reshape(V // 2, 2 * D)
    return _emb_kernel(x, maskf, tag_id, tag_pad.reshape(-1), emb_pairs)


# speedup vs baseline: 2.2703x; 2.2703x over previous
import functools

import jax
import jax.numpy as jnp
from jax import lax
from jax.experimental import pallas as pl
from jax.experimental.pallas import tpu as pltpu
from jax.experimental.pallas import tpu_sc as plsc

N = 16384
V = 1000000
D = 64
SLAB = 8
T_PAD = 64
CH = 32


def _make_kernel():
    info = plsc.get_sparse_core_info()
    NC, NS, L = info.num_cores, info.num_subcores, info.num_lanes
    NW = NC * NS
    BPW = N // NW
    NCH = BPW // CH

    mesh = plsc.VectorSubcoreMesh(core_axis_name="c", subcore_axis_name="s")

    @functools.partial(
        pl.kernel,
        mesh=mesh,
        out_type=jax.ShapeDtypeStruct((N, D), jnp.float32),
        compiler_params=pltpu.CompilerParams(needs_layout_passes=False),
        scratch_types=[
            pltpu.VMEM((BPW,), jnp.int32),
            pltpu.VMEM((BPW,), jnp.int32),
            pltpu.VMEM((CH, SLAB, D), jnp.float32),
            pltpu.VMEM((CH, SLAB, D), jnp.float32),
            pltpu.VMEM((CH, D), jnp.float32),
            pltpu.VMEM((T_PAD * T_PAD,), jnp.float32),
            pltpu.VMEM((T_PAD,), jnp.float32),
            pltpu.VMEM((BPW,), jnp.int32),
            pltpu.VMEM((BPW,), jnp.float32),
            pltpu.VMEM((BPW,), jnp.float32),
            pltpu.SemaphoreType.DMA,
        ],
    )
    def emb_kernel(x_hbm, maskf_hbm, tid_hbm, tag_hbm, emb_hbm, out_hbm,
                   idx_v, sidx_v, slab_a, slab_b, stage_v, tag_v, sums_v,
                   tid_v, maskf_v, scale_v, gsem):
        wid = lax.axis_index("s") * NC + lax.axis_index("c")
        base = wid * BPW

        pltpu.sync_copy(x_hbm.at[pl.ds(base, BPW)], idx_v)
        for g in range(BPW // L):
            sl = pl.ds(g * L, L)
            sidx_v[sl] = lax.shift_right_logical(idx_v[sl], 3)

        bufs = (slab_a, slab_b)

        def fire(c, buf):
            def fire16(g, _):
                iv = sidx_v[pl.ds(c * CH + g * L, L)]
                for r in range(L):
                    pltpu.async_copy(emb_hbm.at[pl.ds(iv[r], 1)],
                                     buf.at[pl.ds(g * L + r, 1)], gsem)
                return _
            lax.fori_loop(0, CH // L, fire16, None)

        fire(0, slab_a)
        fire(1, slab_b)

        pltpu.sync_copy(tag_hbm, tag_v)
        pltpu.sync_copy(tid_hbm.at[pl.ds(base, BPW)], tid_v)
        pltpu.sync_copy(maskf_hbm.at[pl.ds(base, BPW)], maskf_v)

        lanes = lax.iota(jnp.int32, L)
        for g in range(T_PAD // L):
            t_vec = lanes + (g * L)
            row_base = t_vec * T_PAD
            acc = jnp.zeros((L,), jnp.float32)
            for c in range(T_PAD):
                acc = acc + plsc.load_gather(tag_v, [row_base + c])
            plsc.store_scatter(sums_v, [t_vec], acc)

        for g in range(BPW // L):
            sl = pl.ds(g * L, L)
            scale_v[sl] = plsc.load_gather(sums_v, [tid_v[sl]]) * maskf_v[sl]

        for c in range(NCH):
            buf = bufs[c % 2]

            def drain(k, _, buf=buf):
                pltpu.make_async_copy(emb_hbm.at[pl.ds(0, 1)],
                                      buf.at[pl.ds(k, 1)], gsem).wait()
                return _
            lax.fori_loop(0, CH, drain, None)

            def extract(g16, _, c=c, buf=buf):
                n0 = c * CH + g16 * L
                r8v = jnp.bitwise_and(idx_v[pl.ds(n0, L)], 7)
                sv = scale_v[pl.ds(n0, L)]
                for i in range(L):
                    k = g16 * L + i
                    sb = jnp.full((L,), sv[i], jnp.float32)
                    r8 = r8v[i]
                    for j in range(D // L):
                        sl = pl.ds(j * L, L)
                        stage_v[k, sl] = buf[k, r8, sl] * sb
                return _

            lax.fori_loop(0, CH // L, extract, None)
            pltpu.sync_copy(stage_v, out_hbm.at[pl.ds(base + c * CH, CH)])
            if c + 2 < NCH:
                fire(c + 2, buf)

    return emb_kernel


_emb_kernel = _make_kernel()


@jax.jit
def kernel(x, mask, tag_id, emb_table, tag_table):
    x = x.astype(jnp.int32)
    maskf = mask.astype(jnp.float32)
    tag_id = tag_id.astype(jnp.int32)
    t, td = tag_table.shape
    tag_pad = jnp.zeros((T_PAD, T_PAD), jnp.float32).at[:t, :td].set(tag_table)
    emb_slabs = emb_table.reshape(V // SLAB, SLAB, D)
    return _emb_kernel(x, maskf, tag_id, tag_pad.reshape(-1), emb_slabs)
